# Initial kernel scaffold; baseline (speedup 1.0000x reference)
import functools

import jax
import jax.numpy as jnp
from jax import lax
from jax.experimental import pallas as pl
from jax.experimental.pallas import tpu as pltpu
from jax.experimental.pallas import tpu_sc as plsc

N_NODES = 50000
N_EDGES = 800000
DIM_INNER = 128
DIM_EDGE = 16
NUM_EDGE_TYPES = 1000
EPS = 1e-5

NW = 32
NODE_CHUNK = 112
NODE_CHUNKS = 14
NODE_PER_W = NODE_CHUNK * NODE_CHUNKS
NODE_PAD = NODE_PER_W * NW
EDGE_PER_W = 25024
EDGE_PAD = EDGE_PER_W * NW
EDGE_TAB_PAD = 1024
EDGE_CHUNK = 1564
EDGE_CHUNKS = 16
NF = DIM_INNER // 16

_mesh = plsc.VectorSubcoreMesh(core_axis_name="c", subcore_axis_name="s")


@functools.partial(
    pl.kernel,
    mesh=_mesh,
    out_type=[
        jax.ShapeDtypeStruct((NODE_PAD, DIM_INNER), jnp.float32),
        jax.ShapeDtypeStruct((NW, DIM_INNER), jnp.float32),
        jax.ShapeDtypeStruct((NW, DIM_INNER), jnp.float32),
        jax.ShapeDtypeStruct((NW, EDGE_TAB_PAD), jnp.float32),
    ],
    scratch_types=[
        pltpu.VMEM((NODE_CHUNKS, NODE_CHUNK), jnp.int32),
        pltpu.VMEM((NODE_CHUNK, DIM_INNER), jnp.float32),
        pltpu.VMEM((DIM_INNER,), jnp.float32),
        pltpu.VMEM((DIM_INNER,), jnp.float32),
        pltpu.VMEM((EDGE_PER_W,), jnp.int32),
        pltpu.VMEM((EDGE_TAB_PAD,), jnp.float32),
        pltpu.SemaphoreType.DMA,
    ],
)
def _k1(x_hbm, eidx_hbm, tab_hbm, rawh_hbm, psum_hbm, psq_hbm, cnt_hbm,
        nidx_v, rows_v, sum_v, sq_v, eidx_v, cnt_v, sem):
    wid = lax.axis_index("s") * 2 + lax.axis_index("c")
    base = wid * NODE_PER_W
    n_real = jnp.clip(N_NODES - base, 0, NODE_PER_W)

    pltpu.sync_copy(x_hbm.at[wid], nidx_v)
    pltpu.sync_copy(eidx_hbm.at[wid], eidx_v)

    zero16 = jnp.zeros((16,), jnp.float32)
    for i in range(EDGE_TAB_PAD // 16):
        cnt_v[pl.ds(i * 16, 16)] = zero16

    acc = (zero16,) * (2 * NF)
    for c in range(NODE_CHUNKS):
        pltpu.async_copy(tab_hbm.at[nidx_v.at[c]], rows_v, sem).wait()
        pltpu.sync_copy(rows_v, rawh_hbm.at[pl.ds(base + c * NODE_CHUNK,
                                                  NODE_CHUNK)])
        r_lim = jnp.clip(n_real - c * NODE_CHUNK, 0, NODE_CHUNK)

        def row_body(r, a):
            new = []
            for f in range(NF):
                v = rows_v[r, pl.ds(f * 16, 16)]
                new.append(a[f] + v)
            for f in range(NF):
                v = rows_v[r, pl.ds(f * 16, 16)]
                new.append(a[NF + f] + v * v)
            return tuple(new)

        acc = lax.fori_loop(0, r_lim, row_body, acc)

    for f in range(NF):
        sum_v[pl.ds(f * 16, 16)] = acc[f]
        sq_v[pl.ds(f * 16, 16)] = acc[NF + f]
    pltpu.sync_copy(sum_v, psum_hbm.at[wid])
    pltpu.sync_copy(sq_v, psq_hbm.at[wid])

    ones = jnp.full((16,), 1.0, jnp.float32)

    def hist_body(i, carry):
        iv = eidx_v[pl.ds(i * 16, 16)]
        plsc.addupdate_scatter(cnt_v, [iv], ones)
        return carry

    lax.fori_loop(0, EDGE_PER_W // 16, hist_body, 0)
    pltpu.sync_copy(cnt_v, cnt_hbm.at[wid])


def _k2_body(psum, psq, cnt, etab, ng, nb, eg, eb,
             nscale, nshift, etabn):
    nsum = jnp.sum(psum[:], axis=0)
    nsq = jnp.sum(psq[:], axis=0)
    mean = nsum / N_NODES
    var = nsq / N_NODES - mean * mean
    inv = lax.rsqrt(var + EPS)
    sc = ng[:] * inv
    nscale[:] = sc
    nshift[:] = nb[:] - mean * sc

    c = jnp.sum(cnt[:], axis=0)[:, None]
    t = etab[:]
    esum = jnp.sum(t * c, axis=0)
    esq = jnp.sum(t * t * c, axis=0)
    em = esum / N_EDGES
    ev = esq / N_EDGES - em * em
    einv = lax.rsqrt(ev + EPS)
    esc = eg[:] * einv
    esh = eb[:] - em * esc
    etabn[:] = t * esc[None, :] + esh[None, :]


def _k3a_body(raw, scale, shift, out):
    out[:] = raw[:] * scale[:][None, :] + shift[:][None, :]


@functools.partial(
    pl.kernel,
    mesh=_mesh,
    out_type=jax.ShapeDtypeStruct((EDGE_PAD, DIM_EDGE), jnp.float32),
    scratch_types=[
        pltpu.VMEM((EDGE_TAB_PAD, DIM_EDGE), jnp.float32),
        pltpu.VMEM((EDGE_PER_W,), jnp.int32),
        pltpu.VMEM((EDGE_CHUNK, DIM_EDGE), jnp.float32),
        pltpu.SemaphoreType.DMA,
    ],
)
def _k3b(eidx_hbm, etabn_hbm, out_hbm, tab_v, eidx_v, out_v, sem):
    wid = lax.axis_index("s") * 2 + lax.axis_index("c")
    base = wid * EDGE_PER_W
    pltpu.sync_copy(etabn_hbm, tab_v)
    pltpu.sync_copy(eidx_hbm.at[wid], eidx_v)

    for c in range(EDGE_CHUNKS):
        def edge_body(j, carry):
            row = eidx_v[c * EDGE_CHUNK + j]
            out_v[j, pl.ds(0, DIM_EDGE)] = tab_v[row, pl.ds(0, DIM_EDGE)]
            return carry

        lax.fori_loop(0, EDGE_CHUNK, edge_body, 0)
        pltpu.sync_copy(out_v, out_hbm.at[pl.ds(base + c * EDGE_CHUNK,
                                                EDGE_CHUNK)])


def kernel(x, edge_attr, node_table, edge_table, node_gamma, node_beta,
           edge_gamma, edge_beta):
    x_pad = jnp.concatenate(
        [x, jnp.zeros((NODE_PAD - N_NODES,), x.dtype)]
    ).reshape(NW, NODE_CHUNKS, NODE_CHUNK)
    e_pad = jnp.concatenate(
        [edge_attr,
         jnp.full((EDGE_PAD - N_EDGES,), NUM_EDGE_TYPES, edge_attr.dtype)]
    ).reshape(NW, EDGE_PER_W)
    etab_pad = jnp.concatenate(
        [edge_table,
         jnp.zeros((EDGE_TAB_PAD - NUM_EDGE_TYPES, DIM_EDGE),
                   edge_table.dtype)])

    rawh, psum, psq, cnt = _k1(x_pad, e_pad, node_table)

    nscale, nshift, etabn = pl.pallas_call(
        _k2_body,
        out_shape=[
            jax.ShapeDtypeStruct((DIM_INNER,), jnp.float32),
            jax.ShapeDtypeStruct((DIM_INNER,), jnp.float32),
            jax.ShapeDtypeStruct((EDGE_TAB_PAD, DIM_EDGE), jnp.float32),
        ],
    )(psum, psq, cnt, etab_pad, node_gamma, node_beta, edge_gamma, edge_beta)

    rows_per_blk = 448
    h = pl.pallas_call(
        _k3a_body,
        grid=(NODE_PAD // rows_per_blk,),
        in_specs=[
            pl.BlockSpec((rows_per_blk, DIM_INNER), lambda i: (i, 0)),
            pl.BlockSpec((DIM_INNER,), lambda i: (0,)),
            pl.BlockSpec((DIM_INNER,), lambda i: (0,)),
        ],
        out_specs=pl.BlockSpec((rows_per_blk, DIM_INNER), lambda i: (i, 0)),
        out_shape=jax.ShapeDtypeStruct((NODE_PAD, DIM_INNER), jnp.float32),
    )(rawh, nscale, nshift)

    e = _k3b(e_pad, etabn)

    return h[:N_NODES], e[:N_EDGES]


# speedup vs baseline: 2.8792x; 2.8792x over previous
import functools

import jax
import jax.numpy as jnp
from jax import lax
from jax.experimental import pallas as pl
from jax.experimental.pallas import tpu as pltpu
from jax.experimental.pallas import tpu_sc as plsc

N_NODES = 50000
N_EDGES = 800000
DIM_INNER = 128
DIM_EDGE = 16
NUM_EDGE_TYPES = 1000
EPS = 1e-5

NW = 32
NODE_CHUNK = 112
NODE_CHUNKS = 14
NODE_PER_W = NODE_CHUNK * NODE_CHUNKS
NODE_PAD = NODE_PER_W * NW
EDGE_PER_W = 25024
EDGE_PAD = EDGE_PER_W * NW
EDGE_TAB_PAD = 1024
EDGE_CHUNK = 1472
EDGE_CHUNKS = 17
NF = DIM_INNER // 16

_mesh = plsc.VectorSubcoreMesh(core_axis_name="c", subcore_axis_name="s")
_sc_params = pltpu.CompilerParams(needs_layout_passes=False,
                                  use_tc_tiling_on_sc=False)


@functools.partial(
    pl.kernel,
    mesh=_mesh,
    out_type=[
        jax.ShapeDtypeStruct((NODE_PAD, DIM_INNER), jnp.float32),
        jax.ShapeDtypeStruct((NW, DIM_INNER), jnp.float32),
        jax.ShapeDtypeStruct((NW, DIM_INNER), jnp.float32),
        jax.ShapeDtypeStruct((NW, EDGE_TAB_PAD // 16, 16), jnp.float32),
    ],
    scratch_types=[
        pltpu.VMEM((NODE_CHUNKS, NODE_CHUNK), jnp.int32),
        pltpu.VMEM((NODE_CHUNK, DIM_INNER), jnp.float32),
        pltpu.VMEM((DIM_INNER,), jnp.float32),
        pltpu.VMEM((DIM_INNER,), jnp.float32),
        pltpu.VMEM((EDGE_PER_W,), jnp.int32),
        pltpu.VMEM((EDGE_TAB_PAD // 16, 16), jnp.float32),
        pltpu.SemaphoreType.DMA,
    ],
    compiler_params=_sc_params,
)
def _k1(x_hbm, eidx_hbm, tab_hbm, rawh_hbm, psum_hbm, psq_hbm, cnt_hbm,
        nidx_v, rows_v, sum_v, sq_v, eidx_v, cnt_v, sem):
    wid = lax.axis_index("s") * 2 + lax.axis_index("c")
    base = wid * NODE_PER_W
    n_real = jnp.clip(N_NODES - base, 0, NODE_PER_W)

    pltpu.sync_copy(x_hbm.at[wid], nidx_v)
    pltpu.sync_copy(eidx_hbm.at[wid], eidx_v)

    zero16 = jnp.zeros((16,), jnp.float32)
    for i in range(EDGE_TAB_PAD // 16):
        cnt_v[i, pl.ds(0, 16)] = zero16

    acc = (zero16,) * (2 * NF)
    for c in range(NODE_CHUNKS):
        pltpu.async_copy(tab_hbm.at[nidx_v.at[c]], rows_v, sem).wait()
        pltpu.sync_copy(rows_v, rawh_hbm.at[pl.ds(base + c * NODE_CHUNK,
                                                  NODE_CHUNK)])
        r_lim = jnp.clip(n_real - c * NODE_CHUNK, 0, NODE_CHUNK)

        def row_body(r, a):
            new = []
            for f in range(NF):
                v = rows_v[r, pl.ds(f * 16, 16)]
                new.append(a[f] + v)
            for f in range(NF):
                v = rows_v[r, pl.ds(f * 16, 16)]
                new.append(a[NF + f] + v * v)
            return tuple(new)

        acc = lax.fori_loop(0, r_lim, row_body, acc)

    for f in range(NF):
        sum_v[pl.ds(f * 16, 16)] = acc[f]
        sq_v[pl.ds(f * 16, 16)] = acc[NF + f]
    pltpu.sync_copy(sum_v, psum_hbm.at[wid])
    pltpu.sync_copy(sq_v, psq_hbm.at[wid])

    ones = jnp.full((16,), 1.0, jnp.float32)

    def hist_body(i, carry):
        iv = eidx_v[pl.ds(i * 16, 16)]
        plsc.addupdate_scatter(cnt_v, [iv >> 4, iv & 15], ones)
        return carry

    lax.fori_loop(0, EDGE_PER_W // 16, hist_body, 0)
    pltpu.sync_copy(cnt_v, cnt_hbm.at[wid])


def _k2_body(psum, psq, cnt, etab, ng, nb, eg, eb,
             nscale, nshift, etabn):
    nsum = jnp.sum(psum[:], axis=0)
    nsq = jnp.sum(psq[:], axis=0)
    mean = nsum / N_NODES
    var = nsq / N_NODES - mean * mean
    inv = lax.rsqrt(var + EPS)
    sc = ng[:] * inv
    nscale[:] = sc
    nshift[:] = nb[:] - mean * sc

    c = jnp.sum(cnt[:], axis=0)[:, None]
    t = etab[:]
    esum = jnp.sum(t * c, axis=0)
    esq = jnp.sum(t * t * c, axis=0)
    em = esum / N_EDGES
    ev = esq / N_EDGES - em * em
    einv = lax.rsqrt(ev + EPS)
    esc = eg[:] * einv
    esh = eb[:] - em * esc
    etabn[:] = t * esc[None, :] + esh[None, :]


def _k3a_body(raw, scale, shift, out):
    out[:] = raw[:] * scale[:][None, :] + shift[:][None, :]


@functools.partial(
    pl.kernel,
    mesh=_mesh,
    out_type=jax.ShapeDtypeStruct((EDGE_PAD, DIM_EDGE), jnp.float32),
    scratch_types=[
        pltpu.VMEM((EDGE_TAB_PAD, DIM_EDGE), jnp.float32),
        pltpu.VMEM((EDGE_PER_W,), jnp.int32),
        pltpu.VMEM((EDGE_CHUNK, DIM_EDGE), jnp.float32),
        pltpu.SemaphoreType.DMA,
    ],
    compiler_params=_sc_params,
)
def _k3b(eidx_hbm, etabn_hbm, out_hbm, tab_v, eidx_v, out_v, sem):
    wid = lax.axis_index("s") * 2 + lax.axis_index("c")
    base = wid * EDGE_PER_W
    pltpu.sync_copy(etabn_hbm, tab_v)
    pltpu.sync_copy(eidx_hbm.at[wid], eidx_v)

    iota = lax.iota(jnp.int32, 16)
    for c in range(EDGE_CHUNKS):
        def edge_body(i, carry):
            iv = eidx_v[pl.ds(c * EDGE_CHUNK + i * 16, 16)]
            rows = i * 16 + iota
            for f in range(DIM_EDGE):
                fv = jnp.full((16,), f, jnp.int32)
                vals = plsc.load_gather(tab_v, [iv, fv])
                plsc.store_scatter(out_v, [rows, fv], vals)
            return carry

        lax.fori_loop(0, EDGE_CHUNK // 16, edge_body, 0)
        pltpu.sync_copy(out_v, out_hbm.at[pl.ds(base + c * EDGE_CHUNK,
                                                EDGE_CHUNK)])


def kernel(x, edge_attr, node_table, edge_table, node_gamma, node_beta,
           edge_gamma, edge_beta):
    x_pad = jnp.concatenate(
        [x, jnp.zeros((NODE_PAD - N_NODES,), x.dtype)]
    ).reshape(NW, NODE_CHUNKS, NODE_CHUNK)
    e_pad = jnp.concatenate(
        [edge_attr,
         jnp.full((EDGE_PAD - N_EDGES,), NUM_EDGE_TYPES, edge_attr.dtype)]
    ).reshape(NW, EDGE_PER_W)
    etab_pad = jnp.concatenate(
        [edge_table,
         jnp.zeros((EDGE_TAB_PAD - NUM_EDGE_TYPES, DIM_EDGE),
                   edge_table.dtype)])

    rawh, psum, psq, cnt = _k1(x_pad, e_pad, node_table)
    cnt = cnt.reshape(NW, EDGE_TAB_PAD)

    nscale, nshift, etabn = pl.pallas_call(
        _k2_body,
        out_shape=[
            jax.ShapeDtypeStruct((DIM_INNER,), jnp.float32),
            jax.ShapeDtypeStruct((DIM_INNER,), jnp.float32),
            jax.ShapeDtypeStruct((EDGE_TAB_PAD, DIM_EDGE), jnp.float32),
        ],
    )(psum, psq, cnt, etab_pad, node_gamma, node_beta, edge_gamma, edge_beta)

    rows_per_blk = 448
    h = pl.pallas_call(
        _k3a_body,
        grid=(NODE_PAD // rows_per_blk,),
        in_specs=[
            pl.BlockSpec((rows_per_blk, DIM_INNER), lambda i: (i, 0)),
            pl.BlockSpec((DIM_INNER,), lambda i: (0,)),
            pl.BlockSpec((DIM_INNER,), lambda i: (0,)),
        ],
        out_specs=pl.BlockSpec((rows_per_blk, DIM_INNER), lambda i: (i, 0)),
        out_shape=jax.ShapeDtypeStruct((NODE_PAD, DIM_INNER), jnp.float32),
    )(rawh, nscale, nshift)

    e = _k3b(e_pad, etabn)

    return h[:N_NODES], e[:N_EDGES]


# speedup vs baseline: 4.1936x; 1.4566x over previous
import functools

import jax
import jax.numpy as jnp
from jax import lax
from jax.experimental import pallas as pl
from jax.experimental.pallas import tpu as pltpu
from jax.experimental.pallas import tpu_sc as plsc

N_NODES = 50000
N_EDGES = 800000
DIM_INNER = 128
DIM_EDGE = 16
NUM_EDGE_TYPES = 1000
EPS = 1e-5

NW = 32
NODE_CHUNK = 112
NODE_CHUNKS = 14
NODE_PER_W = NODE_CHUNK * NODE_CHUNKS
NBUF = 4
EDGE_PER_W = 25024
EDGE_TAB_PAD = 1024
EDGE_CHUNK = 1472
EDGE_CHUNKS = 17
HIST_GROUPS = EDGE_PER_W // 16
HIST_PER_CHUNK = 112

ROWS_BLK = 2000
N_BLKS = N_NODES // ROWS_BLK

_mesh = plsc.VectorSubcoreMesh(core_axis_name="c", subcore_axis_name="s")
_sc_params = pltpu.CompilerParams(needs_layout_passes=False,
                                  use_tc_tiling_on_sc=False)


@functools.partial(
    pl.kernel,
    mesh=_mesh,
    out_type=[
        jax.ShapeDtypeStruct((N_NODES, DIM_INNER), jnp.float32),
        jax.ShapeDtypeStruct((NW, EDGE_TAB_PAD // 16, 16), jnp.float32),
    ],
    scratch_types=[
        pltpu.VMEM((NODE_PER_W,), jnp.int32),
        pltpu.VMEM((NODE_CHUNK, DIM_INNER), jnp.float32),
        pltpu.VMEM((NODE_CHUNK, DIM_INNER), jnp.float32),
        pltpu.VMEM((NODE_CHUNK, DIM_INNER), jnp.float32),
        pltpu.VMEM((NODE_CHUNK, DIM_INNER), jnp.float32),
        pltpu.VMEM((EDGE_PER_W,), jnp.int32),
        pltpu.VMEM((EDGE_TAB_PAD // 16, 16), jnp.float32),
        pltpu.SemaphoreType.DMA,
        pltpu.SemaphoreType.DMA,
        pltpu.SemaphoreType.DMA,
        pltpu.SemaphoreType.DMA,
        pltpu.SemaphoreType.DMA,
        pltpu.SemaphoreType.DMA,
        pltpu.SemaphoreType.DMA,
        pltpu.SemaphoreType.DMA,
    ],
    compiler_params=_sc_params,
)
def _k1(x_hbm, eidx_hbm, tab_hbm, rawh_hbm, cnt_hbm,
        nidx_v, rb0, rb1, rb2, rb3, eidx_v, cnt_v,
        gs0, gs1, gs2, gs3, ws0, ws1, ws2, ws3):
    wid = lax.axis_index("s") * 2 + lax.axis_index("c")
    nbase = jnp.minimum(wid * NODE_PER_W, N_NODES - NODE_PER_W)
    ebase = jnp.minimum(wid * EDGE_PER_W, N_EDGES - EDGE_PER_W)
    ethr = wid * EDGE_PER_W - ebase

    bufs = (rb0, rb1, rb2, rb3)
    gsems = (gs0, gs1, gs2, gs3)
    wsems = (ws0, ws1, ws2, ws3)

    pltpu.sync_copy(x_hbm.at[pl.ds(nbase, NODE_PER_W)], nidx_v)
    pltpu.sync_copy(eidx_hbm.at[pl.ds(ebase, EDGE_PER_W)], eidx_v)

    zero16 = jnp.zeros((16,), jnp.float32)
    for i in range(EDGE_TAB_PAD // 16):
        cnt_v[i, pl.ds(0, 16)] = zero16

    iota = lax.iota(jnp.int32, 16)
    ethr16 = jnp.full((16,), 0, jnp.int32) + ethr

    g = [None] * NODE_CHUNKS
    w = [None] * NODE_CHUNKS
    for c in range(NBUF):
        g[c] = pltpu.async_copy(
            tab_hbm.at[nidx_v.at[pl.ds(c * NODE_CHUNK, NODE_CHUNK)]],
            bufs[c], gsems[c])

    def hist_body(i, carry):
        iv = eidx_v[pl.ds(i * 16, 16)]
        lpos = i * 16 + iota
        ones = jnp.where(lpos >= ethr16, 1.0, 0.0)
        plsc.addupdate_scatter(cnt_v, [iv >> 4, iv & 15], ones)
        return carry

    for c in range(NODE_CHUNKS):
        g0 = c * HIST_PER_CHUNK
        g1 = min((c + 1) * HIST_PER_CHUNK, HIST_GROUPS)
        lax.fori_loop(g0, g1, hist_body, 0)

        bi = c % NBUF
        g[c].wait()
        w[c] = pltpu.async_copy(
            bufs[bi], rawh_hbm.at[pl.ds(nbase + c * NODE_CHUNK, NODE_CHUNK)],
            wsems[bi])
        n = c + NBUF
        if n < NODE_CHUNKS:
            w[c].wait()
            g[n] = pltpu.async_copy(
                tab_hbm.at[nidx_v.at[pl.ds(n * NODE_CHUNK, NODE_CHUNK)]],
                bufs[bi], gsems[bi])

    for c in range(NODE_CHUNKS - NBUF, NODE_CHUNKS):
        w[c].wait()
    pltpu.sync_copy(cnt_v, cnt_hbm.at[wid])


def _k2_body(rawh, cnt, etab, ng, nb, eg, eb,
             nscale, nshift, etabn, acc):
    i = pl.program_id(0)

    @pl.when(i == 0)
    def _init():
        acc[...] = jnp.zeros((2, DIM_INNER), jnp.float32)

    blk = rawh[...]
    acc[0:1, :] += jnp.sum(blk, axis=0, keepdims=True)
    acc[1:2, :] += jnp.sum(blk * blk, axis=0, keepdims=True)

    @pl.when(i == N_BLKS - 1)
    def _fold():
        mean = acc[0:1, :] / N_NODES
        var = acc[1:2, :] / N_NODES - mean * mean
        inv = lax.rsqrt(var + EPS)
        sc = ng[...][None, :] * inv
        nscale[...] = sc
        nshift[...] = nb[...][None, :] - mean * sc

        crow = jnp.sum(cnt[...], axis=0, keepdims=True)
        t = etab[...]
        esum = jnp.dot(crow, t, preferred_element_type=jnp.float32)
        esq = jnp.dot(crow, t * t, preferred_element_type=jnp.float32)
        em = esum / N_EDGES
        ev = esq / N_EDGES - em * em
        einv = lax.rsqrt(ev + EPS)
        esc = eg[...][None, :] * einv
        esh = eb[...][None, :] - em * esc
        etabn[...] = t * esc + esh


def _k3a_body(raw, scale, shift, out):
    out[...] = raw[...] * scale[...] + shift[...]


@functools.partial(
    pl.kernel,
    mesh=_mesh,
    out_type=jax.ShapeDtypeStruct((N_EDGES, DIM_EDGE), jnp.float32),
    scratch_types=[
        pltpu.VMEM((EDGE_TAB_PAD, DIM_EDGE), jnp.float32),
        pltpu.VMEM((EDGE_PER_W,), jnp.int32),
        pltpu.VMEM((EDGE_CHUNK, DIM_EDGE), jnp.float32),
        pltpu.VMEM((EDGE_CHUNK, DIM_EDGE), jnp.float32),
        pltpu.SemaphoreType.DMA,
        pltpu.SemaphoreType.DMA,
    ],
    compiler_params=_sc_params,
)
def _k3b(eidx_hbm, etabn_hbm, out_hbm, tab_v, eidx_v, ob0, ob1, os0, os1):
    wid = lax.axis_index("s") * 2 + lax.axis_index("c")
    base = jnp.minimum(wid * EDGE_PER_W, N_EDGES - EDGE_PER_W)
    pltpu.sync_copy(etabn_hbm, tab_v)
    pltpu.sync_copy(eidx_hbm.at[pl.ds(base, EDGE_PER_W)], eidx_v)

    obufs = (ob0, ob1)
    osems = (os0, os1)
    iota = lax.iota(jnp.int32, 16)
    w = [None] * EDGE_CHUNKS
    for c in range(EDGE_CHUNKS):
        bi = c % 2
        out_v = obufs[bi]

        def edge_body(i, carry):
            iv = eidx_v[pl.ds(c * EDGE_CHUNK + i * 16, 16)]
            rows = i * 16 + iota
            for f in range(DIM_EDGE):
                fv = jnp.full((16,), f, jnp.int32)
                vals = plsc.load_gather(tab_v, [iv, fv])
                plsc.store_scatter(out_v, [rows, fv], vals)
            return carry

        if c >= 2:
            w[c - 2].wait()
        lax.fori_loop(0, EDGE_CHUNK // 16, edge_body, 0)
        w[c] = pltpu.async_copy(
            out_v, out_hbm.at[pl.ds(base + c * EDGE_CHUNK, EDGE_CHUNK)],
            osems[bi])
    w[EDGE_CHUNKS - 2].wait()
    w[EDGE_CHUNKS - 1].wait()


def kernel(x, edge_attr, node_table, edge_table, node_gamma, node_beta,
           edge_gamma, edge_beta):
    etab_pad = jnp.concatenate(
        [edge_table,
         jnp.zeros((EDGE_TAB_PAD - NUM_EDGE_TYPES, DIM_EDGE),
                   edge_table.dtype)])

    rawh, cnt = _k1(x, edge_attr, node_table)
    cnt = cnt.reshape(NW, EDGE_TAB_PAD)

    nscale, nshift, etabn = pl.pallas_call(
        _k2_body,
        grid=(N_BLKS,),
        in_specs=[
            pl.BlockSpec((ROWS_BLK, DIM_INNER), lambda i: (i, 0)),
            pl.BlockSpec((NW, EDGE_TAB_PAD), lambda i: (0, 0)),
            pl.BlockSpec((EDGE_TAB_PAD, DIM_EDGE), lambda i: (0, 0)),
            pl.BlockSpec((DIM_INNER,), lambda i: (0,)),
            pl.BlockSpec((DIM_INNER,), lambda i: (0,)),
            pl.BlockSpec((DIM_EDGE,), lambda i: (0,)),
            pl.BlockSpec((DIM_EDGE,), lambda i: (0,)),
        ],
        out_specs=[
            pl.BlockSpec((1, DIM_INNER), lambda i: (0, 0)),
            pl.BlockSpec((1, DIM_INNER), lambda i: (0, 0)),
            pl.BlockSpec((EDGE_TAB_PAD, DIM_EDGE), lambda i: (0, 0)),
        ],
        out_shape=[
            jax.ShapeDtypeStruct((1, DIM_INNER), jnp.float32),
            jax.ShapeDtypeStruct((1, DIM_INNER), jnp.float32),
            jax.ShapeDtypeStruct((EDGE_TAB_PAD, DIM_EDGE), jnp.float32),
        ],
        scratch_shapes=[pltpu.VMEM((2, DIM_INNER), jnp.float32)],
    )(rawh, cnt, etab_pad, node_gamma, node_beta, edge_gamma, edge_beta)

    h = pl.pallas_call(
        _k3a_body,
        grid=(N_BLKS,),
        in_specs=[
            pl.BlockSpec((ROWS_BLK, DIM_INNER), lambda i: (i, 0)),
            pl.BlockSpec((1, DIM_INNER), lambda i: (0, 0)),
            pl.BlockSpec((1, DIM_INNER), lambda i: (0, 0)),
        ],
        out_specs=pl.BlockSpec((ROWS_BLK, DIM_INNER), lambda i: (i, 0)),
        out_shape=jax.ShapeDtypeStruct((N_NODES, DIM_INNER), jnp.float32),
    )(rawh, nscale, nshift)

    e = _k3b(edge_attr, etabn)

    return h, e
